# Initial kernel scaffold; baseline (speedup 1.0000x reference)
import jax
import jax.numpy as jnp
from jax.experimental import pallas as pl


def kernel(fs_s_0, fs_t_0, idx, contrast_idx, W, b, memory):
    raise NotImplementedError("write your pallas kernel here")


# speedup vs baseline: 1.7267x; 1.7267x over previous
import functools

import jax
import jax.numpy as jnp
from jax import lax
from jax.experimental import pallas as pl
from jax.experimental.pallas import tpu as pltpu
from jax.experimental.pallas import tpu_sc as plsc

_EPS = 1e-07
_T = 50.0
_NDATA = 100000
_K1 = 513
_KPAD = 528
_B = 1024
_FEAT = 128
_NC, _NS, _LANES = 2, 16, 16
_NW = _NC * _NS
_B_PER_W = _B // _NW
_PAIRS_PER_W = _B_PER_W * _KPAD
_CHUNK = 48
_CHUNKS_PER_B = _KPAD // _CHUNK
_NCHUNKS = _B_PER_W * _CHUNKS_PER_B
_GROUPS = _CHUNK // _LANES
_NBUF = 4


def _embed_body(x_ref, w_ref, b_ref, o_ref):
    o_ref[...] = lax.dot_general(
        x_ref[...], w_ref[...], (((1,), (1,)), ((), ())),
        preferred_element_type=jnp.float32) + b_ref[...]


_embed = pl.pallas_call(
    _embed_body,
    out_shape=jax.ShapeDtypeStruct((_B, _FEAT), jnp.float32),
)


def _sc_body(mem_hbm, idx_hbm, fs_hbm, out_hbm, idx_v, fs_v, d2_v, *bufsems):
    bufs, sems = bufsems[:_NBUF], bufsems[_NBUF:]
    wid = lax.axis_index("s") * _NC + lax.axis_index("c")

    pltpu.sync_copy(idx_hbm.at[pl.ds(wid * _NCHUNKS, _NCHUNKS)], idx_v)
    pltpu.sync_copy(
        fs_hbm.at[pl.ds(wid * _B_PER_W * _FEAT, _B_PER_W * _FEAT)], fs_v)

    lane = lax.iota(jnp.int32, _LANES)
    row_vecs = [lane + g * _LANES for g in range(_GROUPS)]

    def issue(c, k):
        pltpu.async_copy(mem_hbm.at[idx_v.at[c]], bufs[k], sems[k])

    def wait(k):
        pltpu.make_async_copy(mem_hbm.at[idx_v.at[0]], bufs[k], sems[k]).wait()

    def compute(c, k):
        buf = bufs[k]
        fs_off = pl.multiple_of((c // _CHUNKS_PER_B) * _FEAT, _FEAT)
        zero = jnp.zeros((_LANES,), jnp.float32)
        zero_i = jnp.zeros((_LANES,), jnp.int32)

        def body(db, carry):
            accs = list(carry)
            fvec = fs_v[pl.ds(fs_off + db * _LANES, _LANES)]
            dbase = db * _LANES
            for d0 in range(_LANES):
                f = fvec[d0]
                col = zero_i + (dbase + d0)
                for g in range(_GROUPS):
                    m = plsc.load_gather(buf, [row_vecs[g], col])
                    diff = m - f
                    accs[g] = accs[g] + diff * diff
            return tuple(accs)

        a0, a1, a2 = lax.fori_loop(0, _FEAT // _LANES, body,
                                   (zero, zero, zero))
        base = pl.multiple_of(c * _CHUNK, 8)
        for g, acc in enumerate((a0, a1, a2)):
            d2_v[pl.ds(base + g * _LANES, _LANES)] = acc

    for k in range(_NBUF):
        issue(k, k)

    @pl.loop(0, _NCHUNKS, step=_NBUF)
    def _(cc):
        for k in range(_NBUF):
            c = cc + k
            wait(k)
            compute(c, k)
            nxt = c + _NBUF

            @pl.when(nxt < _NCHUNKS)
            def _():
                issue(nxt, k)

    pltpu.sync_copy(d2_v, out_hbm.at[pl.ds(wid * _PAIRS_PER_W, _PAIRS_PER_W)])


_sc_d2 = pl.kernel(
    _sc_body,
    out_type=jax.ShapeDtypeStruct((_NW * _PAIRS_PER_W,), jnp.float32),
    mesh=plsc.VectorSubcoreMesh(core_axis_name="c", subcore_axis_name="s"),
    compiler_params=pltpu.CompilerParams(needs_layout_passes=False),
    scratch_types=(
        [pltpu.VMEM((_NCHUNKS, _CHUNK), jnp.int32),
         pltpu.VMEM((_B_PER_W * _FEAT,), jnp.float32),
         pltpu.VMEM((_PAIRS_PER_W,), jnp.float32)]
        + [pltpu.VMEM((_CHUNK, _FEAT), jnp.float32) for _ in range(_NBUF)]
        + [pltpu.SemaphoreType.DMA for _ in range(_NBUF)]),
)


def _loss_body(d2_ref, o_ref):
    d2 = d2_ref[...]
    col = lax.broadcasted_iota(jnp.int32, (_B, _KPAD), 1)
    valid = col < _K1
    residual = float(_K1 - 1) / float(_NDATA)
    out = jnp.where(valid, jnp.exp(-jnp.sqrt(d2) / _T), 0.0)
    z = jnp.sum(out) * (float(_NDATA) / float(_B * _K1))
    p = out / z
    p_pos = p[:, 0:1]
    log_d1 = jnp.log(p_pos / (p_pos + residual + _EPS))
    neg = jnp.where(valid & (col >= 1),
                    jnp.log(residual / (p + residual + _EPS)), 0.0)
    log_d0 = jnp.sum(neg, axis=1, keepdims=True)
    o_ref[0, 0] = -jnp.mean(log_d1 + log_d0)


_loss = pl.pallas_call(
    _loss_body,
    out_shape=jax.ShapeDtypeStruct((1, 1), jnp.float32),
    out_specs=pl.BlockSpec(memory_space=pltpu.SMEM),
)


def kernel(fs_s_0, fs_t_0, idx, contrast_idx, W, b, memory):
    f_s = _embed(fs_s_0, W, b.reshape(1, _FEAT))
    all_idx = jnp.concatenate([idx[:, None], contrast_idx], axis=1)
    all_idx = jnp.pad(all_idx, ((0, 0), (0, _KPAD - _K1)))
    d2 = _sc_d2(memory[0],
                all_idx.reshape(_NW * _NCHUNKS, _CHUNK).astype(jnp.int32),
                f_s.reshape(-1))
    return _loss(d2.reshape(_B, _KPAD))[0, 0]


# speedup vs baseline: 2.8283x; 1.6380x over previous
import functools

import jax
import jax.numpy as jnp
from jax import lax
from jax.experimental import pallas as pl
from jax.experimental.pallas import tpu as pltpu
from jax.experimental.pallas import tpu_sc as plsc

_EPS = 1e-07
_T = 50.0
_NDATA = 100000
_K1 = 513
_KPAD = 528
_B = 1024
_FEAT = 128
_NC, _NS, _LANES = 2, 16, 16
_NW = _NC * _NS
_B_PER_W = _B // _NW
_PAIRS_PER_W = _B_PER_W * _KPAD
_CHUNK = 48
_CHUNKS_PER_B = _KPAD // _CHUNK
_NCHUNKS = _B_PER_W * _CHUNKS_PER_B
_RU = 4
_NBUF = 4


def _embed_body(x_ref, w_ref, b_ref, o_ref):
    o_ref[...] = lax.dot_general(
        x_ref[...], w_ref[...], (((1,), (1,)), ((), ())),
        preferred_element_type=jnp.float32) + b_ref[...]


_embed = pl.pallas_call(
    _embed_body,
    out_shape=jax.ShapeDtypeStruct((_B, _FEAT), jnp.float32),
)


def _sc_body(mem_hbm, idx_hbm, fs_hbm, out_hbm, idx_v, fs_v, d2_v, *bufsems):
    bufs, sems = bufsems[:_NBUF], bufsems[_NBUF:]
    wid = lax.axis_index("s") * _NC + lax.axis_index("c")

    pltpu.sync_copy(idx_hbm.at[pl.ds(wid * _NCHUNKS, _NCHUNKS)], idx_v)
    pltpu.sync_copy(
        fs_hbm.at[pl.ds(wid * _B_PER_W * _FEAT, _B_PER_W * _FEAT)], fs_v)

    def issue(c, k):
        pltpu.async_copy(mem_hbm.at[idx_v.at[c]], bufs[k], sems[k])

    def wait(k):
        pltpu.make_async_copy(mem_hbm.at[idx_v.at[0]], bufs[k], sems[k]).wait()

    lane = lax.iota(jnp.int32, _LANES)
    masks = [lane == u for u in range(_LANES)]

    def compute(c, k):
        buf = bufs[k]
        fs_off = pl.multiple_of((c // _CHUNKS_PER_B) * _FEAT, _FEAT)
        f = [fs_v[pl.ds(fs_off + d * _LANES, _LANES)]
             for d in range(_FEAT // _LANES)]
        base = pl.multiple_of(c * _CHUNK, 8)

        @pl.loop(0, _CHUNK // _LANES)
        def _(rr):
            r0 = rr * _LANES
            out = jnp.zeros((_LANES,), jnp.float32)
            for u in range(_LANES):
                r = r0 + u
                diff = buf[r, pl.ds(0, _LANES)] - f[0]
                acc = diff * diff
                for d in range(1, _FEAT // _LANES):
                    diff = buf[r, pl.ds(d * _LANES, _LANES)] - f[d]
                    acc = acc + diff * diff
                out = jnp.where(masks[u], jnp.sum(acc), out)
            d2_v[pl.ds(base + r0, _LANES)] = out

    for k in range(_NBUF):
        issue(k, k)

    @pl.loop(0, _NCHUNKS, step=_NBUF)
    def _(cc):
        for k in range(_NBUF):
            c = cc + k
            wait(k)
            compute(c, k)
            nxt = c + _NBUF

            @pl.when(nxt < _NCHUNKS)
            def _():
                issue(nxt, k)

    pltpu.sync_copy(d2_v, out_hbm.at[pl.ds(wid * _PAIRS_PER_W, _PAIRS_PER_W)])


_sc_d2 = pl.kernel(
    _sc_body,
    out_type=jax.ShapeDtypeStruct((_NW * _PAIRS_PER_W,), jnp.float32),
    mesh=plsc.VectorSubcoreMesh(core_axis_name="c", subcore_axis_name="s"),
    compiler_params=pltpu.CompilerParams(needs_layout_passes=False),
    scratch_types=(
        [pltpu.VMEM((_NCHUNKS, _CHUNK), jnp.int32),
         pltpu.VMEM((_B_PER_W * _FEAT,), jnp.float32),
         pltpu.VMEM((_PAIRS_PER_W,), jnp.float32)]
        + [pltpu.VMEM((_CHUNK, _FEAT), jnp.float32) for _ in range(_NBUF)]
        + [pltpu.SemaphoreType.DMA for _ in range(_NBUF)]),
)


def _loss_body(d2_ref, o_ref):
    d2 = d2_ref[...]
    col = lax.broadcasted_iota(jnp.int32, (_B, _KPAD), 1)
    valid = col < _K1
    residual = float(_K1 - 1) / float(_NDATA)
    out = jnp.where(valid, jnp.exp(-jnp.sqrt(d2) / _T), 0.0)
    z = jnp.sum(out) * (float(_NDATA) / float(_B * _K1))
    p = out / z
    p_pos = p[:, 0:1]
    log_d1 = jnp.log(p_pos / (p_pos + residual + _EPS))
    neg = jnp.where(valid & (col >= 1),
                    jnp.log(residual / (p + residual + _EPS)), 0.0)
    log_d0 = jnp.sum(neg, axis=1, keepdims=True)
    o_ref[0, 0] = -jnp.mean(log_d1 + log_d0)


_loss = pl.pallas_call(
    _loss_body,
    out_shape=jax.ShapeDtypeStruct((1, 1), jnp.float32),
    out_specs=pl.BlockSpec(memory_space=pltpu.SMEM),
)


def kernel(fs_s_0, fs_t_0, idx, contrast_idx, W, b, memory):
    f_s = _embed(fs_s_0, W, b.reshape(1, _FEAT))
    all_idx = jnp.concatenate([idx[:, None], contrast_idx], axis=1)
    all_idx = jnp.pad(all_idx, ((0, 0), (0, _KPAD - _K1)))
    d2 = _sc_d2(memory[0],
                all_idx.reshape(_NW * _NCHUNKS, _CHUNK).astype(jnp.int32),
                f_s.reshape(-1))
    return _loss(d2.reshape(_B, _KPAD))[0, 0]


# speedup vs baseline: 11.1745x; 3.9509x over previous
import functools

import jax
import jax.numpy as jnp
from jax import lax
from jax.experimental import pallas as pl
from jax.experimental.pallas import tpu as pltpu
from jax.experimental.pallas import tpu_sc as plsc

_EPS = 1e-07
_T = 50.0
_NDATA = 100000
_K = 512
_B = 1024
_FEAT = 128
_NC, _NS, _LANES = 2, 16, 16
_NW = _NC * _NS
_B_PER_W = _B // _NW
_CHUNK = 128
_CHUNKS_PER_B = _K // _CHUNK
_NCHUNKS = _B_PER_W * _CHUNKS_PER_B
_NEG_PER_W = _B_PER_W * _K
_OUT_PER_W = _NEG_PER_W + _B_PER_W
_DGRP = _FEAT // _LANES
_NBUF = 4


def _embed_body(x_ref, w_ref, b_ref, o_ref):
    o_ref[...] = lax.dot_general(
        x_ref[...], w_ref[...], (((1,), (1,)), ((), ())),
        preferred_element_type=jnp.float32) + b_ref[...]


_embed = pl.pallas_call(
    _embed_body,
    out_shape=jax.ShapeDtypeStruct((_B, _FEAT), jnp.float32),
)


def _sc_body(mem_hbm, idxn_hbm, idxp_hbm, fs_hbm, out_hbm,
             idx_v, idxp_v, fs_v, d2_v, pbuf, psem, *bufsems):
    bufs, sems = bufsems[:_NBUF], bufsems[_NBUF:]
    wid = lax.axis_index("s") * _NC + lax.axis_index("c")

    pltpu.sync_copy(idxn_hbm.at[pl.ds(wid * _NCHUNKS, _NCHUNKS)], idx_v)
    pltpu.sync_copy(idxp_hbm.at[pl.ds(wid * _B_PER_W, _B_PER_W)], idxp_v)
    pltpu.sync_copy(
        fs_hbm.at[pl.ds(wid * _B_PER_W * _FEAT, _B_PER_W * _FEAT)], fs_v)

    def issue(c, k):
        pltpu.async_copy(mem_hbm.at[idx_v.at[c]], bufs[k], sems[k])

    def wait(k):
        pltpu.make_async_copy(mem_hbm.at[idx_v.at[0]], bufs[k], sems[k]).wait()

    lane = lax.iota(jnp.int32, _LANES)
    masks = [lane == u for u in range(_LANES)]

    pltpu.async_copy(mem_hbm.at[idxp_v], pbuf, psem)
    for k in range(_NBUF):
        issue(k, k)

    def compute(c, k):
        buf = bufs[k]
        fs_off = pl.multiple_of((c // _CHUNKS_PER_B) * _FEAT, _FEAT)
        f = [fs_v[pl.ds(fs_off + d * _LANES, _LANES)] for d in range(_DGRP)]
        base = pl.multiple_of(c * _CHUNK, 8)

        @pl.loop(0, _CHUNK // _LANES)
        def _(rr):
            r0 = rr * _LANES
            out = jnp.zeros((_LANES,), jnp.float32)
            for u in range(_LANES):
                r = r0 + u
                diff = buf[r, pl.ds(0, _LANES)] - f[0]
                acc = diff * diff
                for d in range(1, _DGRP):
                    diff = buf[r, pl.ds(d * _LANES, _LANES)] - f[d]
                    acc = acc + diff * diff
                out = jnp.where(masks[u], jnp.sum(acc), out)
            d2_v[pl.ds(base + r0, _LANES)] = out

    @pl.loop(0, _NCHUNKS, step=_NBUF)
    def _(cc):
        for k in range(_NBUF):
            c = cc + k
            wait(k)
            compute(c, k)
            nxt = c + _NBUF

            @pl.when(nxt < _NCHUNKS)
            def _():
                issue(nxt, k)

    pltpu.make_async_copy(mem_hbm.at[idxp_v], pbuf, psem).wait()

    @pl.loop(0, _B_PER_W // _LANES)
    def _(rr):
        r0 = rr * _LANES
        out = jnp.zeros((_LANES,), jnp.float32)
        for u in range(_LANES):
            r = r0 + u
            foff = r * _FEAT
            diff = pbuf[r, pl.ds(0, _LANES)] - fs_v[pl.ds(foff, _LANES)]
            acc = diff * diff
            for d in range(1, _DGRP):
                diff = (pbuf[r, pl.ds(d * _LANES, _LANES)]
                        - fs_v[pl.ds(foff + d * _LANES, _LANES)])
                acc = acc + diff * diff
            out = jnp.where(masks[u], jnp.sum(acc), out)
        d2_v[pl.ds(_NEG_PER_W + r0, _LANES)] = out

    pltpu.sync_copy(d2_v, out_hbm.at[pl.ds(wid * _OUT_PER_W, _OUT_PER_W)])


_sc_d2 = pl.kernel(
    _sc_body,
    out_type=jax.ShapeDtypeStruct((_NW * _OUT_PER_W,), jnp.float32),
    mesh=plsc.VectorSubcoreMesh(core_axis_name="c", subcore_axis_name="s"),
    compiler_params=pltpu.CompilerParams(needs_layout_passes=False),
    scratch_types=(
        [pltpu.VMEM((_NCHUNKS, _CHUNK), jnp.int32),
         pltpu.VMEM((_B_PER_W,), jnp.int32),
         pltpu.VMEM((_B_PER_W * _FEAT,), jnp.float32),
         pltpu.VMEM((_OUT_PER_W,), jnp.float32),
         pltpu.VMEM((_B_PER_W, _FEAT), jnp.float32),
         pltpu.SemaphoreType.DMA]
        + [pltpu.VMEM((_CHUNK, _FEAT), jnp.float32) for _ in range(_NBUF)]
        + [pltpu.SemaphoreType.DMA for _ in range(_NBUF)]),
)


def _loss_body(d2n_ref, d2p_ref, o_ref):
    d2n = d2n_ref[...]
    d2p = d2p_ref[...]
    residual = float(_K) / float(_NDATA)
    out_n = jnp.exp(-jnp.sqrt(d2n) / _T)
    out_p = jnp.exp(-jnp.sqrt(d2p) / _T)
    z = ((jnp.sum(out_n) + jnp.sum(out_p))
         * (float(_NDATA) / float(_B * (_K + 1))))
    p_pos = out_p / z
    p_neg = out_n / z
    log_d1 = jnp.log(p_pos / (p_pos + residual + _EPS))
    log_d0 = jnp.sum(jnp.log(residual / (p_neg + residual + _EPS)),
                     axis=1, keepdims=True)
    o_ref[0, 0] = -jnp.mean(log_d1 + log_d0)


_loss = pl.pallas_call(
    _loss_body,
    out_shape=jax.ShapeDtypeStruct((1, 1), jnp.float32),
    out_specs=pl.BlockSpec(memory_space=pltpu.SMEM),
)


def kernel(fs_s_0, fs_t_0, idx, contrast_idx, W, b, memory):
    f_s = _embed(fs_s_0, W, b.reshape(1, _FEAT))
    d2 = _sc_d2(memory[0],
                contrast_idx.reshape(_NW * _NCHUNKS, _CHUNK).astype(jnp.int32),
                idx.astype(jnp.int32),
                f_s.reshape(-1))
    d2 = d2.reshape(_NW, _OUT_PER_W)
    d2_neg = d2[:, :_NEG_PER_W].reshape(_B, _K)
    d2_pos = d2[:, _NEG_PER_W:].reshape(_B, 1)
    return _loss(d2_neg, d2_pos)[0, 0]
